# Initial kernel scaffold; baseline (speedup 1.0000x reference)
import jax
import jax.numpy as jnp
from jax.experimental import pallas as pl


def kernel(x, edge_index, W1, b1, W2, b2, W3, b3, W4, b4):
    raise NotImplementedError("write your pallas kernel here")


# speedup vs baseline: 9.7334x; 9.7334x over previous
import functools

import jax
import jax.numpy as jnp
from jax import lax
from jax.experimental import pallas as pl
from jax.experimental.pallas import tpu as pltpu
from jax.experimental.pallas import tpu_sc as plsc

_N = 10000
_NP = 10240
_E = 320000
_CH = 128
_NCH = _E // _CH
_NC = 2
_NS = 16
_NW = _NC * _NS
_D = 128
_K2 = 16
_BM = 1024
_RPT = _NP // _NS

_CHUNKS_BASE = _NCH // _NW
_CHUNKS_REM = _NCH % _NW

_f32 = jnp.float32
_HIGH = jax.lax.Precision.HIGHEST


def _sc_mesh():
  return plsc.VectorSubcoreMesh(core_axis_name="c", subcore_axis_name="s")


def _make_segsum_kernel(k, stage_p):
  stage_scratch = [pltpu.VMEM_SHARED((_NP, k), _f32)] if stage_p else []

  @functools.partial(
      pl.kernel,
      out_type=jax.ShapeDtypeStruct((2 * _NP, k), _f32),
      mesh=_sc_mesh(),
      compiler_params=(pltpu.CompilerParams(use_tc_tiling_on_sc=False)
                       if k < 128 else None),
      scratch_types=[
          pltpu.VMEM_SHARED((_NP, k), _f32),
          *stage_scratch,
          pltpu.VMEM((2, _CH), jnp.int32),
          pltpu.VMEM((2, _CH), jnp.int32),
          pltpu.VMEM((2, _CH, k), _f32),
          pltpu.SemaphoreType.DMA((2,)),
          pltpu.SemaphoreType.DMA((2,)),
          pltpu.SemaphoreType.DMA((2,)),
          pltpu.SemaphoreType.DMA((2,)),
      ],
  )
  def segsum_kernel(p, src2, dst2, zeros2, out, acc, *rest):
    if stage_p:
      pstage, srcv, dstv, rows, sis, sid_, sg, ss = rest
    else:
      srcv, dstv, rows, sis, sid_, sg, ss = rest
      pstage = p
    c = lax.axis_index("c")
    s = lax.axis_index("s")
    wid = s * _NC + c

    pltpu.sync_copy(zeros2.at[pl.ds(s * _RPT, _RPT)],
                    acc.at[pl.ds(s * _RPT, _RPT)])
    if stage_p:
      pltpu.sync_copy(p.at[pl.ds(s * _RPT, _RPT)],
                      pstage.at[pl.ds(s * _RPT, _RPT)])
    plsc.subcore_barrier()

    n = _CHUNKS_BASE + jnp.where(wid < _CHUNKS_REM, 1, 0)

    def body(t, carry):
      b = lax.rem(t, 2)
      j = wid + t * _NW

      @pl.when(t >= 2)
      def _():
        pltpu.make_async_copy(rows.at[b], acc.at[dstv.at[b]], ss.at[b]).wait()

      cs = pltpu.async_copy(src2.at[j], srcv.at[b], sis.at[b])
      cd = pltpu.async_copy(dst2.at[j], dstv.at[b], sid_.at[b])
      cs.wait()
      cd.wait()
      pltpu.async_copy(pstage.at[srcv.at[b]], rows.at[b], sg.at[b]).wait()
      pltpu.async_copy(rows.at[b], acc.at[dstv.at[b]], ss.at[b], add=True)
      return carry

    lax.fori_loop(0, n, body, 0)
    for b in range(2):
      pltpu.make_async_copy(rows.at[b], acc.at[dstv.at[b]], ss.at[b]).wait()
    plsc.subcore_barrier()

    pltpu.sync_copy(acc.at[pl.ds(s * _RPT, _RPT)],
                    out.at[pl.ds(c * _NP + s * _RPT, _RPT)])

  return segsum_kernel


def _norms_body(do0_ref, do1_ref, di0_ref, di1_ref, out_ref):
  do = jnp.sum(do0_ref[...] + do1_ref[...], axis=-1) * (1.0 / _K2)
  di = jnp.sum(di0_ref[...] + di1_ref[...], axis=-1) * (1.0 / _K2)
  out_ref[0, :] = lax.rsqrt(jnp.maximum(do, 1.0))
  out_ref[1, :] = lax.rsqrt(jnp.maximum(di, 1.0))


def _tc_first_body(norm2_ref, x_ref, w_ref, out_ref):
  norm_o = norm2_ref[0, :]
  t = jnp.dot(x_ref[...], w_ref[...], preferred_element_type=_f32,
              precision=_HIGH)
  out_ref[...] = t * norm_o[:, None]


def _tc_mid_body(norm2_ref, a0_ref, a1_ref, b_ref, w_ref, out_ref):
  norm_o = norm2_ref[0, :]
  norm_i = norm2_ref[1, :]
  agg = a0_ref[...] + a1_ref[...]
  h = jnp.maximum(agg * norm_i[:, None] + b_ref[0, :], 0.0)
  t = jnp.dot(h, w_ref[...], preferred_element_type=_f32, precision=_HIGH)
  out_ref[...] = t * norm_o[:, None]


def _tc_last_body(norm2_ref, a0_ref, a1_ref, b_ref, out_ref):
  norm_i = norm2_ref[1, :]
  agg = a0_ref[...] + a1_ref[...]
  out_ref[...] = agg * norm_i[:, None] + b_ref[0, :]


_GRID = (_NP // _BM,)


def _deg_spec():
  return pl.BlockSpec((2, _BM), lambda i: (0, i))


def _tc_norms(dego, degi):
  sp = _partial_specs(_K2)
  return pl.pallas_call(
      _norms_body,
      grid=_GRID,
      in_specs=[sp[0], sp[1], sp[0], sp[1]],
      out_specs=pl.BlockSpec((2, _BM), lambda i: (0, i)),
      out_shape=jax.ShapeDtypeStruct((2, _NP), _f32),
  )(dego, dego, degi, degi)


def _row_spec(kw):
  return pl.BlockSpec((_BM, kw), lambda i: (i, 0))


def _partial_specs(kw):
  return [pl.BlockSpec((_BM, kw), lambda i: (i, 0)),
          pl.BlockSpec((_BM, kw), lambda i: (i + _NP // _BM, 0))]


def _full_spec(shape):
  return pl.BlockSpec(shape, lambda i: tuple(0 for _ in shape))


def _tc_first(deg4, x, w):
  return pl.pallas_call(
      _tc_first_body,
      grid=_GRID,
      in_specs=[_deg_spec(), _row_spec(_D), _full_spec(w.shape)],
      out_specs=_row_spec(w.shape[1]),
      out_shape=jax.ShapeDtypeStruct((_NP, w.shape[1]), _f32),
  )(deg4, x, w)


def _tc_mid(deg4, a, b, w):
  kin, kout = w.shape
  s0, s1 = _partial_specs(kin)
  return pl.pallas_call(
      _tc_mid_body,
      grid=_GRID,
      in_specs=[_deg_spec(), s0, s1, _full_spec((1, kin)), _full_spec(w.shape)],
      out_specs=_row_spec(kout),
      out_shape=jax.ShapeDtypeStruct((_NP, kout), _f32),
  )(deg4, a, a, b, w)


def _tc_last(deg4, a, b):
  kin = a.shape[1]
  s0, s1 = _partial_specs(kin)
  return pl.pallas_call(
      _tc_last_body,
      grid=_GRID,
      in_specs=[_deg_spec(), s0, s1, _full_spec((1, kin))],
      out_specs=_row_spec(kin),
      out_shape=jax.ShapeDtypeStruct((_NP, kin), _f32),
  )(deg4, a, a, b)


def kernel(x, edge_index, W1, b1, W2, b2, W3, b3, W4, b4):
  src2 = edge_index[0].astype(jnp.int32).reshape(_NCH, _CH)
  dst2 = edge_index[1].astype(jnp.int32).reshape(_NCH, _CH)
  x_pad = jnp.pad(x, ((0, _NP - _N), (0, 0)))

  zeros_d = jnp.zeros((_NP, _D), _f32)
  zeros_k2 = jnp.zeros((_NP, _K2), _f32)

  W3p = jnp.pad(W3, ((0, 0), (0, _K2 - W3.shape[1])))
  b3p = jnp.pad(b3, (0, _K2 - b3.shape[0]))
  W4p = jnp.pad(W4, ((0, _K2 - W4.shape[0]), (0, 0)))

  seg_d = _make_segsum_kernel(_D, stage_p=False)
  seg_k2 = _make_segsum_kernel(_K2, stage_p=True)

  ones16 = jnp.ones((_NP, _K2), _f32)
  dego = seg_k2(ones16, dst2, src2, zeros_k2)
  degi = seg_k2(ones16, src2, dst2, zeros_k2)
  deg4 = _tc_norms(dego, degi)

  p1 = _tc_first(deg4, x_pad, W1)
  a1 = seg_d(p1, src2, dst2, zeros_d)
  p2 = _tc_mid(deg4, a1, b1.reshape(1, -1), W2)
  a2 = seg_d(p2, src2, dst2, zeros_d)
  p3 = _tc_mid(deg4, a2, b2.reshape(1, -1), W3p)
  a3 = seg_k2(p3, src2, dst2, zeros_k2)
  p4 = _tc_mid(deg4, a3, b3p.reshape(1, -1), W4p)
  a4 = seg_k2(p4, src2, dst2, zeros_k2)
  out = _tc_last(deg4, a4, b4.reshape(1, -1))

  return out[:_N]


# speedup vs baseline: 13.8786x; 1.4259x over previous
import functools

import jax
import jax.numpy as jnp
from jax import lax
from jax.experimental import pallas as pl
from jax.experimental.pallas import tpu as pltpu
from jax.experimental.pallas import tpu_sc as plsc

_N = 10000
_NP = 10240
_E = 320000
_CH = 128
_NCH = _E // _CH
_NC = 2
_NS = 16
_NW = _NC * _NS
_D = 128
_K2 = 16
_BM = 1024
_RPT = _NP // _NS

_NB = 80
_EP = _NW * _NB * _CH

_f32 = jnp.float32
_HIGH = jax.lax.Precision.HIGHEST


def _sc_mesh():
  return plsc.VectorSubcoreMesh(core_axis_name="c", subcore_axis_name="s")


def _make_segsum_kernel(k, stage_p):
  stage_scratch = [pltpu.VMEM_SHARED((_NP, k), _f32)] if stage_p else []

  @functools.partial(
      pl.kernel,
      out_type=jax.ShapeDtypeStruct((2 * _NP, k), _f32),
      mesh=_sc_mesh(),
      compiler_params=pltpu.CompilerParams(use_tc_tiling_on_sc=False),
      scratch_types=[
          pltpu.VMEM_SHARED((_NP, k), _f32),
          *stage_scratch,
          pltpu.VMEM((4, _CH), jnp.int32),
          pltpu.VMEM((_NB, _CH), jnp.int32),
          pltpu.VMEM((2, _CH, k), _f32),
          pltpu.SemaphoreType.DMA((4,)),
          pltpu.SemaphoreType.DMA((1,)),
          pltpu.SemaphoreType.DMA((2,)),
          pltpu.SemaphoreType.DMA((2,)),
      ],
  )
  def segsum_kernel(p, src2, dst2, zeros2, out, acc, *rest):
    if stage_p:
      pstage, srcv, dstv, rows, sr, si, sg, ss = rest
    else:
      srcv, dstv, rows, sr, si, sg, ss = rest
      pstage = p
    c = lax.axis_index("c")
    s = lax.axis_index("s")
    wid = s * _NC + c

    cd = pltpu.async_copy(dst2.at[wid], dstv, si.at[0])
    for j in range(3):
      pltpu.async_copy(src2.at[wid, j], srcv.at[j], sr.at[j])
    pltpu.sync_copy(zeros2.at[pl.ds(s * _RPT, _RPT)],
                    acc.at[pl.ds(s * _RPT, _RPT)])
    if stage_p:
      pltpu.sync_copy(p.at[pl.ds(s * _RPT, _RPT)],
                      pstage.at[pl.ds(s * _RPT, _RPT)])
    cd.wait()
    plsc.subcore_barrier()

    pltpu.make_async_copy(src2.at[wid, 0], srcv.at[0], sr.at[0]).wait()
    pltpu.async_copy(pstage.at[srcv.at[0]], rows.at[0], sg.at[0])

    def body(t, carry):
      b = lax.rem(t, 2)

      @pl.when(t + 3 < _NB)
      def _():
        jn = lax.rem(t + 3, 4)
        pltpu.async_copy(src2.at[wid, t + 3], srcv.at[jn], sr.at[jn])

      pltpu.make_async_copy(pstage.at[srcv.at[0]], rows.at[b], sg.at[b]).wait()
      pltpu.async_copy(rows.at[b], acc.at[dstv.at[t]], ss.at[b], add=True)

      @pl.when(t + 1 < _NB)
      def _():
        bn = lax.rem(t + 1, 2)

        @pl.when(t >= 1)
        def _():
          pltpu.make_async_copy(rows.at[bn], acc.at[dstv.at[0]],
                                ss.at[bn]).wait()
        jn = lax.rem(t + 1, 4)
        pltpu.make_async_copy(src2.at[wid, 0], srcv.at[jn], sr.at[jn]).wait()
        pltpu.async_copy(pstage.at[srcv.at[jn]], rows.at[bn], sg.at[bn])
      return carry

    lax.fori_loop(0, _NB, body, 0)
    for b in range(2):
      pltpu.make_async_copy(rows.at[b], acc.at[dstv.at[0]], ss.at[b]).wait()
    plsc.subcore_barrier()

    pltpu.sync_copy(acc.at[pl.ds(s * _RPT, _RPT)],
                    out.at[pl.ds(c * _NP + s * _RPT, _RPT)])

  return segsum_kernel


def _norms_body(do0_ref, do1_ref, di0_ref, di1_ref, out_ref):
  do = jnp.sum(do0_ref[...] + do1_ref[...], axis=-1) * (1.0 / _K2)
  di = jnp.sum(di0_ref[...] + di1_ref[...], axis=-1) * (1.0 / _K2)
  out_ref[0, :] = lax.rsqrt(jnp.maximum(do, 1.0))
  out_ref[1, :] = lax.rsqrt(jnp.maximum(di, 1.0))


def _tc_first_body(norm2_ref, x_ref, w_ref, out_ref):
  norm_o = norm2_ref[0, :]
  t = jnp.dot(x_ref[...], w_ref[...], preferred_element_type=_f32,
              precision=_HIGH)
  out_ref[...] = t * norm_o[:, None]


def _tc_mid_body(norm2_ref, a0_ref, a1_ref, b_ref, w_ref, out_ref):
  norm_o = norm2_ref[0, :]
  norm_i = norm2_ref[1, :]
  agg = a0_ref[...] + a1_ref[...]
  h = jnp.maximum(agg * norm_i[:, None] + b_ref[0, :], 0.0)
  t = jnp.dot(h, w_ref[...], preferred_element_type=_f32, precision=_HIGH)
  out_ref[...] = t * norm_o[:, None]


def _tc_last_body(norm2_ref, a0_ref, a1_ref, b_ref, out_ref):
  norm_i = norm2_ref[1, :]
  agg = a0_ref[...] + a1_ref[...]
  out_ref[...] = agg * norm_i[:, None] + b_ref[0, :]


_GRID = (_NP // _BM,)


def _deg_spec():
  return pl.BlockSpec((2, _BM), lambda i: (0, i))


def _tc_norms(dego, degi):
  sp = _partial_specs(_K2)
  return pl.pallas_call(
      _norms_body,
      grid=_GRID,
      in_specs=[sp[0], sp[1], sp[0], sp[1]],
      out_specs=pl.BlockSpec((2, _BM), lambda i: (0, i)),
      out_shape=jax.ShapeDtypeStruct((2, _NP), _f32),
  )(dego, dego, degi, degi)


def _row_spec(kw):
  return pl.BlockSpec((_BM, kw), lambda i: (i, 0))


def _partial_specs(kw):
  return [pl.BlockSpec((_BM, kw), lambda i: (i, 0)),
          pl.BlockSpec((_BM, kw), lambda i: (i + _NP // _BM, 0))]


def _full_spec(shape):
  return pl.BlockSpec(shape, lambda i: tuple(0 for _ in shape))


def _tc_first(deg4, x, w):
  return pl.pallas_call(
      _tc_first_body,
      grid=_GRID,
      in_specs=[_deg_spec(), _row_spec(_D), _full_spec(w.shape)],
      out_specs=_row_spec(w.shape[1]),
      out_shape=jax.ShapeDtypeStruct((_NP, w.shape[1]), _f32),
  )(deg4, x, w)


def _tc_mid(deg4, a, b, w):
  kin, kout = w.shape
  s0, s1 = _partial_specs(kin)
  return pl.pallas_call(
      _tc_mid_body,
      grid=_GRID,
      in_specs=[_deg_spec(), s0, s1, _full_spec((1, kin)), _full_spec(w.shape)],
      out_specs=_row_spec(kout),
      out_shape=jax.ShapeDtypeStruct((_NP, kout), _f32),
  )(deg4, a, a, b, w)


def _tc_last(deg4, a, b):
  kin = a.shape[1]
  s0, s1 = _partial_specs(kin)
  return pl.pallas_call(
      _tc_last_body,
      grid=_GRID,
      in_specs=[_deg_spec(), s0, s1, _full_spec((1, kin))],
      out_specs=_row_spec(kin),
      out_shape=jax.ShapeDtypeStruct((_NP, kin), _f32),
  )(deg4, a, a, b)


def kernel(x, edge_index, W1, b1, W2, b2, W3, b3, W4, b4):
  pad_idx = (_N + (jnp.arange(_EP - _E, dtype=jnp.int32) % (_NP - _N)))
  src2 = jnp.concatenate([edge_index[0].astype(jnp.int32), pad_idx])
  dst2 = jnp.concatenate([edge_index[1].astype(jnp.int32), pad_idx])
  src2 = src2.reshape(_NW, _NB, _CH)
  dst2 = dst2.reshape(_NW, _NB, _CH)
  x_pad = jnp.pad(x, ((0, _NP - _N), (0, 0)))

  zeros_d = jnp.zeros((_NP, _D), _f32)
  zeros_k2 = jnp.zeros((_NP, _K2), _f32)

  W3p = jnp.pad(W3, ((0, 0), (0, _K2 - W3.shape[1])))
  b3p = jnp.pad(b3, (0, _K2 - b3.shape[0]))
  W4p = jnp.pad(W4, ((0, _K2 - W4.shape[0]), (0, 0)))

  seg_d = _make_segsum_kernel(_D, stage_p=False)
  seg_k2 = _make_segsum_kernel(_K2, stage_p=True)

  ones16 = jnp.ones((_NP, _K2), _f32)
  dego = seg_k2(ones16, dst2, src2, zeros_k2)
  degi = seg_k2(ones16, src2, dst2, zeros_k2)
  deg4 = _tc_norms(dego, degi)

  p1 = _tc_first(deg4, x_pad, W1)
  a1 = seg_d(p1, src2, dst2, zeros_d)
  p2 = _tc_mid(deg4, a1, b1.reshape(1, -1), W2)
  a2 = seg_d(p2, src2, dst2, zeros_d)
  p3 = _tc_mid(deg4, a2, b2.reshape(1, -1), W3p)
  a3 = seg_k2(p3, src2, dst2, zeros_k2)
  p4 = _tc_mid(deg4, a3, b3p.reshape(1, -1), W4p)
  a4 = seg_k2(p4, src2, dst2, zeros_k2)
  out = _tc_last(deg4, a4, b4.reshape(1, -1))

  return out[:_N]


# speedup vs baseline: 16.0249x; 1.1546x over previous
import functools

import jax
import jax.numpy as jnp
from jax import lax
from jax.experimental import pallas as pl
from jax.experimental.pallas import tpu as pltpu
from jax.experimental.pallas import tpu_sc as plsc

_N = 10000
_NP = 10240
_E = 320000
_CH = 128
_NCH = _E // _CH
_NC = 2
_NS = 16
_NW = _NC * _NS
_D = 128
_K2 = 16
_BM = 1024
_RPT = _NP // _NS

_NB = 80
_EP = _NW * _NB * _CH

_f32 = jnp.float32
_HIGH = jax.lax.Precision.HIGHEST


def _sc_mesh():
  return plsc.VectorSubcoreMesh(core_axis_name="c", subcore_axis_name="s")


def _make_segsum_kernel(k, stage_p, ch):
  stage_scratch = [pltpu.VMEM_SHARED((_NP, k), _f32)] if stage_p else []
  nb = _EP // (_NW * ch)

  @functools.partial(
      pl.kernel,
      out_type=jax.ShapeDtypeStruct((2 * _NP, k), _f32),
      mesh=_sc_mesh(),
      compiler_params=pltpu.CompilerParams(use_tc_tiling_on_sc=False),
      scratch_types=[
          pltpu.VMEM_SHARED((_NP, k), _f32),
          *stage_scratch,
          pltpu.VMEM((6, ch), jnp.int32),
          pltpu.VMEM((nb, ch), jnp.int32),
          pltpu.VMEM((4, ch, k), _f32),
          pltpu.SemaphoreType.DMA((6,)),
          pltpu.SemaphoreType.DMA((1,)),
          pltpu.SemaphoreType.DMA((4,)),
          pltpu.SemaphoreType.DMA((4,)),
      ],
  )
  def segsum_kernel(p, src2, dst2, zeros2, out, acc, *rest):
    if stage_p:
      pstage, srcv, dstv, rows, sr, si, sg, ss = rest
    else:
      srcv, dstv, rows, sr, si, sg, ss = rest
      pstage = p
    c = lax.axis_index("c")
    s = lax.axis_index("s")
    wid = s * _NC + c

    cd = pltpu.async_copy(dst2.at[wid], dstv, si.at[0])
    for j in range(5):
      pltpu.async_copy(src2.at[wid, j], srcv.at[j], sr.at[j])
    pltpu.sync_copy(zeros2.at[pl.ds(s * _RPT, _RPT)],
                    acc.at[pl.ds(s * _RPT, _RPT)])
    if stage_p:
      pltpu.sync_copy(p.at[pl.ds(s * _RPT, _RPT)],
                      pstage.at[pl.ds(s * _RPT, _RPT)])
    cd.wait()
    plsc.subcore_barrier()

    for j in range(3):
      pltpu.make_async_copy(src2.at[wid, 0], srcv.at[j], sr.at[j]).wait()
      pltpu.async_copy(pstage.at[srcv.at[j]], rows.at[j], sg.at[j])

    def body(t, carry):
      b = lax.rem(t, 4)

      @pl.when(t + 5 < nb)
      def _():
        jn = lax.rem(t + 5, 6)
        pltpu.async_copy(src2.at[wid, t + 5], srcv.at[jn], sr.at[jn])

      pltpu.make_async_copy(pstage.at[srcv.at[0]], rows.at[b], sg.at[b]).wait()
      pltpu.async_copy(rows.at[b], acc.at[dstv.at[t]], ss.at[b], add=True)

      @pl.when(t + 3 < nb)
      def _():
        bn = lax.rem(t + 3, 4)

        @pl.when(t >= 1)
        def _():
          pltpu.make_async_copy(rows.at[bn], acc.at[dstv.at[0]],
                                ss.at[bn]).wait()
        jn = lax.rem(t + 3, 6)
        pltpu.make_async_copy(src2.at[wid, 0], srcv.at[jn], sr.at[jn]).wait()
        pltpu.async_copy(pstage.at[srcv.at[jn]], rows.at[bn], sg.at[bn])
      return carry

    lax.fori_loop(0, nb, body, 0)
    for b in range(4):
      pltpu.make_async_copy(rows.at[b], acc.at[dstv.at[0]], ss.at[b]).wait()
    plsc.subcore_barrier()

    pltpu.sync_copy(acc.at[pl.ds(s * _RPT, _RPT)],
                    out.at[pl.ds(c * _NP + s * _RPT, _RPT)])

  return segsum_kernel


def _norms_body(do0_ref, do1_ref, di0_ref, di1_ref, out_ref):
  do = jnp.sum(do0_ref[...] + do1_ref[...], axis=-1) * (1.0 / _K2)
  di = jnp.sum(di0_ref[...] + di1_ref[...], axis=-1) * (1.0 / _K2)
  out_ref[0, :] = lax.rsqrt(jnp.maximum(do, 1.0))
  out_ref[1, :] = lax.rsqrt(jnp.maximum(di, 1.0))


def _tc_first_body(norm2_ref, x_ref, w_ref, out_ref):
  norm_o = norm2_ref[0, :]
  t = jnp.dot(x_ref[...], w_ref[...], preferred_element_type=_f32,
              precision=_HIGH)
  out_ref[...] = t * norm_o[:, None]


def _tc_mid_body(norm2_ref, a0_ref, a1_ref, b_ref, w_ref, out_ref):
  norm_o = norm2_ref[0, :]
  norm_i = norm2_ref[1, :]
  agg = a0_ref[...] + a1_ref[...]
  h = jnp.maximum(agg * norm_i[:, None] + b_ref[0, :], 0.0)
  t = jnp.dot(h, w_ref[...], preferred_element_type=_f32, precision=_HIGH)
  out_ref[...] = t * norm_o[:, None]


def _tc_last_body(norm2_ref, a0_ref, a1_ref, b_ref, out_ref):
  norm_i = norm2_ref[1, :]
  agg = a0_ref[...] + a1_ref[...]
  out_ref[...] = agg * norm_i[:, None] + b_ref[0, :]


_GRID = (_NP // _BM,)


def _deg_spec():
  return pl.BlockSpec((2, _BM), lambda i: (0, i))


def _tc_norms(dego, degi):
  sp = _partial_specs(_K2)
  return pl.pallas_call(
      _norms_body,
      grid=_GRID,
      in_specs=[sp[0], sp[1], sp[0], sp[1]],
      out_specs=pl.BlockSpec((2, _BM), lambda i: (0, i)),
      out_shape=jax.ShapeDtypeStruct((2, _NP), _f32),
  )(dego, dego, degi, degi)


def _row_spec(kw):
  return pl.BlockSpec((_BM, kw), lambda i: (i, 0))


def _partial_specs(kw):
  return [pl.BlockSpec((_BM, kw), lambda i: (i, 0)),
          pl.BlockSpec((_BM, kw), lambda i: (i + _NP // _BM, 0))]


def _full_spec(shape):
  return pl.BlockSpec(shape, lambda i: tuple(0 for _ in shape))


def _tc_first(deg4, x, w):
  return pl.pallas_call(
      _tc_first_body,
      grid=_GRID,
      in_specs=[_deg_spec(), _row_spec(_D), _full_spec(w.shape)],
      out_specs=_row_spec(w.shape[1]),
      out_shape=jax.ShapeDtypeStruct((_NP, w.shape[1]), _f32),
  )(deg4, x, w)


def _tc_mid(deg4, a, b, w):
  kin, kout = w.shape
  s0, s1 = _partial_specs(kin)
  return pl.pallas_call(
      _tc_mid_body,
      grid=_GRID,
      in_specs=[_deg_spec(), s0, s1, _full_spec((1, kin)), _full_spec(w.shape)],
      out_specs=_row_spec(kout),
      out_shape=jax.ShapeDtypeStruct((_NP, kout), _f32),
  )(deg4, a, a, b, w)


def _tc_last(deg4, a, b):
  kin = a.shape[1]
  s0, s1 = _partial_specs(kin)
  return pl.pallas_call(
      _tc_last_body,
      grid=_GRID,
      in_specs=[_deg_spec(), s0, s1, _full_spec((1, kin))],
      out_specs=_row_spec(kin),
      out_shape=jax.ShapeDtypeStruct((_NP, kin), _f32),
  )(deg4, a, a, b)


def kernel(x, edge_index, W1, b1, W2, b2, W3, b3, W4, b4):
  pad_idx = (_N + (jnp.arange(_EP - _E, dtype=jnp.int32) % (_NP - _N)))
  srcf = jnp.concatenate([edge_index[0].astype(jnp.int32), pad_idx])
  dstf = jnp.concatenate([edge_index[1].astype(jnp.int32), pad_idx])
  src_w = srcf.reshape(_NW, -1, 64)
  dst_w = dstf.reshape(_NW, -1, 64)
  src_n = srcf.reshape(_NW, -1, _CH)
  dst_n = dstf.reshape(_NW, -1, _CH)
  x_pad = jnp.pad(x, ((0, _NP - _N), (0, 0)))

  zeros_d = jnp.zeros((_NP, _D), _f32)
  zeros_k2 = jnp.zeros((_NP, _K2), _f32)

  W3p = jnp.pad(W3, ((0, 0), (0, _K2 - W3.shape[1])))
  b3p = jnp.pad(b3, (0, _K2 - b3.shape[0]))
  W4p = jnp.pad(W4, ((0, _K2 - W4.shape[0]), (0, 0)))

  seg_d = _make_segsum_kernel(_D, stage_p=False, ch=64)
  seg_k2 = _make_segsum_kernel(_K2, stage_p=True, ch=_CH)

  ones16 = jnp.ones((_NP, _K2), _f32)
  dego = seg_k2(ones16, dst_n, src_n, zeros_k2)
  degi = seg_k2(ones16, src_n, dst_n, zeros_k2)
  deg4 = _tc_norms(dego, degi)

  p1 = _tc_first(deg4, x_pad, W1)
  a1 = seg_d(p1, src_w, dst_w, zeros_d)
  p2 = _tc_mid(deg4, a1, b1.reshape(1, -1), W2)
  a2 = seg_d(p2, src_w, dst_w, zeros_d)
  p3 = _tc_mid(deg4, a2, b2.reshape(1, -1), W3p)
  a3 = seg_k2(p3, src_n, dst_n, zeros_k2)
  p4 = _tc_mid(deg4, a3, b3p.reshape(1, -1), W4p)
  a4 = seg_k2(p4, src_n, dst_n, zeros_k2)
  out = _tc_last(deg4, a4, b4.reshape(1, -1))

  return out[:_N]


# speedup vs baseline: 16.7668x; 1.0463x over previous
import functools

import jax
import jax.numpy as jnp
from jax import lax
from jax.experimental import pallas as pl
from jax.experimental.pallas import tpu as pltpu
from jax.experimental.pallas import tpu_sc as plsc

_N = 10000
_NP = 10240
_E = 320000
_CH = 128
_NCH = _E // _CH
_NC = 2
_NS = 16
_NW = _NC * _NS
_D = 128
_K2 = 16
_BM = 2048
_RPT = _NP // _NS

_NB = 80
_EP = _NW * _NB * _CH

_f32 = jnp.float32
_HIGH = jax.lax.Precision.HIGHEST


def _sc_mesh():
  return plsc.VectorSubcoreMesh(core_axis_name="c", subcore_axis_name="s")


def _make_segsum_kernel(k, stage_p, ch):
  stage_scratch = [pltpu.VMEM_SHARED((_NP, k), _f32)] if stage_p else []
  nb = _EP // (_NW * ch)

  @functools.partial(
      pl.kernel,
      out_type=jax.ShapeDtypeStruct((2 * _NP, k), _f32),
      mesh=_sc_mesh(),
      compiler_params=pltpu.CompilerParams(use_tc_tiling_on_sc=False),
      scratch_types=[
          pltpu.VMEM_SHARED((_NP, k), _f32),
          *stage_scratch,
          pltpu.VMEM((6, ch), jnp.int32),
          pltpu.VMEM((nb, ch), jnp.int32),
          pltpu.VMEM((4, ch, k), _f32),
          pltpu.SemaphoreType.DMA((6,)),
          pltpu.SemaphoreType.DMA((1,)),
          pltpu.SemaphoreType.DMA((4,)),
          pltpu.SemaphoreType.DMA((4,)),
      ],
  )
  def segsum_kernel(p, src2, dst2, zeros2, out, acc, *rest):
    if stage_p:
      pstage, srcv, dstv, rows, sr, si, sg, ss = rest
    else:
      srcv, dstv, rows, sr, si, sg, ss = rest
      pstage = p
    c = lax.axis_index("c")
    s = lax.axis_index("s")
    wid = s * _NC + c

    cd = pltpu.async_copy(dst2.at[wid], dstv, si.at[0])
    for j in range(5):
      pltpu.async_copy(src2.at[wid, j], srcv.at[j], sr.at[j])
    pltpu.sync_copy(zeros2.at[pl.ds(s * _RPT, _RPT)],
                    acc.at[pl.ds(s * _RPT, _RPT)])
    if stage_p:
      pltpu.sync_copy(p.at[pl.ds(s * _RPT, _RPT)],
                      pstage.at[pl.ds(s * _RPT, _RPT)])
    cd.wait()
    plsc.subcore_barrier()

    for j in range(3):
      pltpu.make_async_copy(src2.at[wid, 0], srcv.at[j], sr.at[j]).wait()
      pltpu.async_copy(pstage.at[srcv.at[j]], rows.at[j], sg.at[j])

    def body(t, carry):
      b = lax.rem(t, 4)

      @pl.when(t + 5 < nb)
      def _():
        jn = lax.rem(t + 5, 6)
        pltpu.async_copy(src2.at[wid, t + 5], srcv.at[jn], sr.at[jn])

      pltpu.make_async_copy(pstage.at[srcv.at[0]], rows.at[b], sg.at[b]).wait()
      pltpu.async_copy(rows.at[b], acc.at[dstv.at[t]], ss.at[b], add=True)

      @pl.when(t + 3 < nb)
      def _():
        bn = lax.rem(t + 3, 4)

        @pl.when(t >= 1)
        def _():
          pltpu.make_async_copy(rows.at[bn], acc.at[dstv.at[0]],
                                ss.at[bn]).wait()
        jn = lax.rem(t + 3, 6)
        pltpu.make_async_copy(src2.at[wid, 0], srcv.at[jn], sr.at[jn]).wait()
        pltpu.async_copy(pstage.at[srcv.at[jn]], rows.at[bn], sg.at[bn])
      return carry

    lax.fori_loop(0, nb, body, 0)
    for b in range(4):
      pltpu.make_async_copy(rows.at[b], acc.at[dstv.at[0]], ss.at[b]).wait()
    plsc.subcore_barrier()

    pltpu.sync_copy(acc.at[pl.ds(s * _RPT, _RPT)],
                    out.at[pl.ds(c * _NP + s * _RPT, _RPT)])

  return segsum_kernel


def _tc_first_body(do0_ref, do1_ref, di0_ref, di1_ref, x_ref, w_ref,
                   norm2_ref, out_ref):
  do = jnp.sum(do0_ref[...] + do1_ref[...], axis=-1) * (1.0 / _K2)
  di = jnp.sum(di0_ref[...] + di1_ref[...], axis=-1) * (1.0 / _K2)
  norm_o = lax.rsqrt(jnp.maximum(do, 1.0))
  norm2_ref[0, :] = norm_o
  norm2_ref[1, :] = lax.rsqrt(jnp.maximum(di, 1.0))
  t = jnp.dot(x_ref[...], w_ref[...], preferred_element_type=_f32,
              precision=_HIGH)
  out_ref[...] = t * norm_o[:, None]


def _tc_mid_body(norm2_ref, a0_ref, a1_ref, b_ref, w_ref, out_ref):
  norm_o = norm2_ref[0, :]
  norm_i = norm2_ref[1, :]
  agg = a0_ref[...] + a1_ref[...]
  h = jnp.maximum(agg * norm_i[:, None] + b_ref[0, :], 0.0)
  t = jnp.dot(h, w_ref[...], preferred_element_type=_f32, precision=_HIGH)
  out_ref[...] = t * norm_o[:, None]


def _tc_last_body(norm2_ref, a0_ref, a1_ref, b_ref, out_ref):
  norm_i = norm2_ref[1, :]
  agg = a0_ref[...] + a1_ref[...]
  out_ref[...] = agg * norm_i[:, None] + b_ref[0, :]


_GRID = (_NP // _BM,)


def _deg_spec():
  return pl.BlockSpec((2, _BM), lambda i: (0, i))


def _row_spec(kw):
  return pl.BlockSpec((_BM, kw), lambda i: (i, 0))


def _partial_specs(kw):
  return [pl.BlockSpec((_BM, kw), lambda i: (i, 0)),
          pl.BlockSpec((_BM, kw), lambda i: (i + _NP // _BM, 0))]


def _full_spec(shape):
  return pl.BlockSpec(shape, lambda i: tuple(0 for _ in shape))


def _tc_first(dego, degi, x, w):
  sp = _partial_specs(_K2)
  return pl.pallas_call(
      _tc_first_body,
      grid=_GRID,
      in_specs=[sp[0], sp[1], sp[0], sp[1], _row_spec(_D), _full_spec(w.shape)],
      out_specs=[pl.BlockSpec((2, _BM), lambda i: (0, i)),
                 _row_spec(w.shape[1])],
      out_shape=[jax.ShapeDtypeStruct((2, _NP), _f32),
                 jax.ShapeDtypeStruct((_NP, w.shape[1]), _f32)],
  )(dego, dego, degi, degi, x, w)


def _tc_mid(deg4, a, b, w):
  kin, kout = w.shape
  s0, s1 = _partial_specs(kin)
  return pl.pallas_call(
      _tc_mid_body,
      grid=_GRID,
      in_specs=[_deg_spec(), s0, s1, _full_spec((1, kin)), _full_spec(w.shape)],
      out_specs=_row_spec(kout),
      out_shape=jax.ShapeDtypeStruct((_NP, kout), _f32),
  )(deg4, a, a, b, w)


def _tc_last(deg4, a, b):
  kin = a.shape[1]
  s0, s1 = _partial_specs(kin)
  return pl.pallas_call(
      _tc_last_body,
      grid=_GRID,
      in_specs=[_deg_spec(), s0, s1, _full_spec((1, kin))],
      out_specs=_row_spec(kin),
      out_shape=jax.ShapeDtypeStruct((_NP, kin), _f32),
  )(deg4, a, a, b)


def kernel(x, edge_index, W1, b1, W2, b2, W3, b3, W4, b4):
  pad_idx = (_N + (jnp.arange(_EP - _E, dtype=jnp.int32) % (_NP - _N)))
  srcf = jnp.concatenate([edge_index[0].astype(jnp.int32), pad_idx])
  dstf = jnp.concatenate([edge_index[1].astype(jnp.int32), pad_idx])
  src_w = srcf.reshape(_NW, -1, 64)
  dst_w = dstf.reshape(_NW, -1, 64)
  src_n = srcf.reshape(_NW, -1, _CH)
  dst_n = dstf.reshape(_NW, -1, _CH)
  x_pad = jnp.pad(x, ((0, _NP - _N), (0, 0)))

  zeros_d = jnp.zeros((_NP, _D), _f32)
  zeros_k2 = jnp.zeros((_NP, _K2), _f32)

  W3p = jnp.pad(W3, ((0, 0), (0, _K2 - W3.shape[1])))
  b3p = jnp.pad(b3, (0, _K2 - b3.shape[0]))
  W4p = jnp.pad(W4, ((0, _K2 - W4.shape[0]), (0, 0)))

  seg_d = _make_segsum_kernel(_D, stage_p=False, ch=64)
  seg_k2 = _make_segsum_kernel(_K2, stage_p=True, ch=_CH)

  ones16 = jnp.ones((_NP, _K2), _f32)
  dego = seg_k2(ones16, dst_n, src_n, zeros_k2)
  degi = seg_k2(ones16, src_n, dst_n, zeros_k2)
  deg4, p1 = _tc_first(dego, degi, x_pad, W1)
  a1 = seg_d(p1, src_w, dst_w, zeros_d)
  p2 = _tc_mid(deg4, a1, b1.reshape(1, -1), W2)
  a2 = seg_d(p2, src_w, dst_w, zeros_d)
  p3 = _tc_mid(deg4, a2, b2.reshape(1, -1), W3p)
  a3 = seg_k2(p3, src_n, dst_n, zeros_k2)
  p4 = _tc_mid(deg4, a3, b3p.reshape(1, -1), W4p)
  a4 = seg_k2(p4, src_n, dst_n, zeros_k2)
  out = _tc_last(deg4, a4, b4.reshape(1, -1))

  return out[:_N]
